# Initial kernel scaffold; baseline (speedup 1.0000x reference)
import jax
import jax.numpy as jnp
from jax.experimental import pallas as pl


def kernel(ids, kernel):
    raise NotImplementedError("write your pallas kernel here")


# speedup vs baseline: 1.2075x; 1.2075x over previous
import jax
import jax.numpy as jnp
from jax import lax
from jax.experimental import pallas as pl
from jax.experimental.pallas import tpu as pltpu
from jax.experimental.pallas import tpu_sc as plsc

_N_IDS = 425984
_NUM_WORKERS = 32
_B_PER_W = _N_IDS // _NUM_WORKERS


def _gather_body(ids_hbm, table_hbm, out_hbm, idx_v, vals_v, sem):
    wid = lax.axis_index("s") * 2 + lax.axis_index("c")
    base = wid * _B_PER_W
    pltpu.sync_copy(ids_hbm.at[pl.ds(base, _B_PER_W)], idx_v)
    pltpu.async_copy(table_hbm.at[idx_v], vals_v, sem).wait()
    pltpu.sync_copy(vals_v, out_hbm.at[pl.ds(base, _B_PER_W)])


def kernel(ids, kernel):
    mesh = plsc.VectorSubcoreMesh(core_axis_name="c", subcore_axis_name="s")
    gathered = pl.kernel(
        _gather_body,
        mesh=mesh,
        out_type=jax.ShapeDtypeStruct((_N_IDS,), jnp.float32),
        scratch_types=[
            pltpu.VMEM((_B_PER_W,), jnp.int32),
            pltpu.VMEM((_B_PER_W,), jnp.float32),
            pltpu.SemaphoreType.DMA,
        ],
    )(ids, kernel)
    return gathered[:, None]


# speedup vs baseline: 1.4085x; 1.1665x over previous
import jax
import jax.numpy as jnp
from jax import lax
from jax.experimental import pallas as pl
from jax.experimental.pallas import tpu as pltpu
from jax.experimental.pallas import tpu_sc as plsc

_N_IDS = 425984
_N_TABLE = 1000000
_NUM_WORKERS = 32
_B_PER_W = _N_IDS // _NUM_WORKERS
_T_CHUNK = 31256
_T_LAST = _N_TABLE - 31 * _T_CHUNK


def _gather_body(ids_hbm, table_hbm, out_hbm, idx_v, vals_v, tbuf_v, table_sp,
                 sem):
    sid = lax.axis_index("s")
    wid = sid * 2 + lax.axis_index("c")
    base = wid * _B_PER_W
    pltpu.sync_copy(ids_hbm.at[pl.ds(base, _B_PER_W)], idx_v)
    for r in range(2):
        k = sid * 2 + r
        tbase = k * _T_CHUNK

        @pl.when(k < 31)
        def _():
            pltpu.sync_copy(table_hbm.at[pl.ds(tbase, _T_CHUNK)], tbuf_v)
            pltpu.sync_copy(tbuf_v, table_sp.at[pl.ds(tbase, _T_CHUNK)])

        @pl.when(k == 31)
        def _():
            pltpu.sync_copy(table_hbm.at[pl.ds(tbase, _T_LAST)],
                            tbuf_v.at[pl.ds(0, _T_LAST)])
            pltpu.sync_copy(tbuf_v.at[pl.ds(0, _T_LAST)],
                            table_sp.at[pl.ds(tbase, _T_LAST)])

    plsc.subcore_barrier()
    pltpu.async_copy(table_sp.at[idx_v], vals_v, sem).wait()
    pltpu.sync_copy(vals_v, out_hbm.at[pl.ds(base, _B_PER_W)])


def kernel(ids, kernel):
    mesh = plsc.VectorSubcoreMesh(core_axis_name="c", subcore_axis_name="s")
    gathered = pl.kernel(
        _gather_body,
        mesh=mesh,
        out_type=jax.ShapeDtypeStruct((_N_IDS,), jnp.float32),
        scratch_types=[
            pltpu.VMEM((_B_PER_W,), jnp.int32),
            pltpu.VMEM((_B_PER_W,), jnp.float32),
            pltpu.VMEM((_T_CHUNK,), jnp.float32),
            pltpu.VMEM_SHARED((_N_TABLE,), jnp.float32),
            pltpu.SemaphoreType.DMA,
        ],
    )(ids, kernel)
    return gathered[:, None]
